# baseline (speedup 1.0000x reference)
import functools

import numpy as np
import jax
import jax.numpy as jnp
from jax import lax
from jax.experimental import pallas as pl
from jax.experimental.pallas import tpu as pltpu
from jax.experimental.pallas import tpu_sc as plsc

B = 16384
L = 200
D = 64
GS = ((0, 104), (104, 96))
NC = 2
NS = 16
NW = NC * NS
BH = B // 2
BPW = BH // NW
CB = 16
NCHUNK = BPW // CB
VECS = D // 16


NBUF = 4
LK = NBUF - 1


def _pool_body(x_hbm, emb_hbm, out_hbm, idxA, idxB, rows0, rows1, rows2, rows3,
               outA, outB, sem0, sem1, sem2, sem3, semi):
    wid = lax.axis_index("s") * NC + lax.axis_index("c")
    base = wid * BPW
    rows = (rows0, rows1, rows2, rows3)
    sems = (sem0, sem1, sem2, sem3)

    def idx_copy(c, dst):
        c = jnp.minimum(c, NCHUNK - 1)
        return pltpu.async_copy(
            x_hbm.at[pl.ds(base + c * CB, CB)], dst, semi)

    def issue(idx_ref, b, buf):
        for off, sz in GS:
            pltpu.async_copy(
                emb_hbm.at[idx_ref.at[b, pl.ds(off, sz)]],
                rows[buf].at[pl.ds(off, sz)],
                sems[buf],
            )

    def wait_row(buf):
        for off, sz in GS:
            pltpu.make_async_copy(
                emb_hbm.at[pl.ds(0, sz)],
                rows[buf].at[pl.ds(off, sz)],
                sems[buf],
            ).wait()

    def reduce_row(out_v, b, buf):
        r = rows[buf]
        un = 8
        himask = jnp.full((16,), -65536, jnp.int32)

        def rbody(i, acc):
            acc = list(acc)
            for u in range(un):
                row = i * un + u
                p = (u % 2) * 4
                for d in range(2):
                    vi = plsc.bitcast(r[row, pl.ds(d * 32, 32)], jnp.int32)
                    lo = plsc.bitcast(lax.shift_left(vi, 16), jnp.float32)
                    hi = plsc.bitcast(vi & himask, jnp.float32)
                    acc[p + d] = acc[p + d] + lo
                    acc[p + d + 2] = acc[p + d + 2] + hi
            return tuple(acc)

        zero = jnp.zeros((16,), jnp.float32)
        acc = plsc.parallel_loop(0, L // un, carry=tuple(zero for _ in range(8)))(
            lambda i, a: rbody(i, a))
        for d in range(4):
            out_v[b, pl.ds(d * 16, 16)] = (acc[d] + acc[d + 4]) * (1.0 / L)

    def phase(cur_idx, nxt_idx, c_cur, idx_handle, out_v):
        for b in range(CB):
            t = b + LK
            if t < CB:
                issue(cur_idx, t, t % NBUF)
            else:
                if t == CB:
                    idx_handle.wait()
                issue(nxt_idx, t - CB, t % NBUF)
            wait_row(b % NBUF)
            reduce_row(out_v, b, b % NBUF)
        pltpu.sync_copy(out_v, out_hbm.at[pl.ds((base + c_cur * CB), CB)])

    def pair_body(k, carry):
        hB = idx_copy(2 * k + 1, idxB)
        phase(idxA, idxB, 2 * k, hB, outA)
        hA = idx_copy(2 * k + 2, idxA)
        phase(idxB, idxA, 2 * k + 1, hA, outB)
        return carry

    pltpu.sync_copy(x_hbm.at[pl.ds(base, CB)], idxA)
    for p in range(LK):
        issue(idxA, p, p % NBUF)

    lax.fori_loop(0, NCHUNK // 2, pair_body, 0)

    for p in range(LK):
        wait_row(p % NBUF)


def _pool(x, emb):
    mesh = plsc.VectorSubcoreMesh(core_axis_name="c", subcore_axis_name="s")
    return pl.kernel(
        _pool_body,
        out_type=jax.ShapeDtypeStruct((BH, D), jnp.float32),
        mesh=mesh,
        compiler_params=pltpu.CompilerParams(use_tc_tiling_on_sc=False, needs_layout_passes=False),
        scratch_types=[
            pltpu.VMEM((CB, L), jnp.int32),
            pltpu.VMEM((CB, L), jnp.int32),
            pltpu.VMEM((L, D), jnp.bfloat16),
            pltpu.VMEM((L, D), jnp.bfloat16),
            pltpu.VMEM((L, D), jnp.bfloat16),
            pltpu.VMEM((L, D), jnp.bfloat16),
            pltpu.VMEM((CB, D), jnp.float32),
            pltpu.VMEM((CB, D), jnp.float32),
            pltpu.SemaphoreType.DMA,
            pltpu.SemaphoreType.DMA,
            pltpu.SemaphoreType.DMA,
            pltpu.SemaphoreType.DMA,
            pltpu.SemaphoreType.DMA,
        ],
    )(x, emb)


def _mlp_body(h_ref, w1_ref, b1_ref, w2_ref, b2_ref, o_ref):
    h = h_ref[...]
    z = jnp.dot(h, w1_ref[...], preferred_element_type=jnp.float32) + b1_ref[...]
    z = jnp.maximum(z, 0.0)
    y = jnp.dot(z, w2_ref[...], preferred_element_type=jnp.float32) + b2_ref[...]
    m = jnp.max(y, axis=1, keepdims=True)
    lse = m + jnp.log(jnp.sum(jnp.exp(y - m), axis=1, keepdims=True))
    o_ref[...] = y - lse


def _mlp(h, W1, b1, W2, b2):
    bm = 2048
    d_h = W1.shape[1]
    d_out = W2.shape[1]
    return pl.pallas_call(
        _mlp_body,
        grid=(BH // bm,),
        in_specs=[
            pl.BlockSpec((bm, D), lambda i: (i, 0)),
            pl.BlockSpec((D, d_h), lambda i: (0, 0)),
            pl.BlockSpec((1, d_h), lambda i: (0, 0)),
            pl.BlockSpec((d_h, d_out), lambda i: (0, 0)),
            pl.BlockSpec((1, d_out), lambda i: (0, 0)),
        ],
        out_specs=pl.BlockSpec((bm, d_out), lambda i: (i, 0)),
        out_shape=jax.ShapeDtypeStruct((BH, d_out), jnp.float32),
    )(h, W1, b1, W2, b2)


_COL_PERM = np.concatenate([
    np.arange(0, 32, 2), np.arange(32, 64, 2),
    np.arange(1, 32, 2), np.arange(33, 64, 2),
])


def kernel(x, emb, W1, b1, W2, b2):
    emb_bf = emb.astype(jnp.bfloat16)
    w1p = W1[_COL_PERM]
    b1r = b1.reshape(1, -1)
    b2r = b2.reshape(1, -1)
    halves = []
    for h in range(2):
        pooled = _pool(lax.slice_in_dim(x, h * (B // 2), (h + 1) * (B // 2)), emb_bf)
        halves.append(_mlp(pooled, w1p, b1r, W2, b2r))
    return jnp.concatenate(halves, axis=0)


# speedup vs baseline: 1.0632x; 1.0632x over previous
import functools

import numpy as np
import jax
import jax.numpy as jnp
from jax import lax
from jax.experimental import pallas as pl
from jax.experimental.pallas import tpu as pltpu
from jax.experimental.pallas import tpu_sc as plsc

B = 16384
L = 200
D = 64
GS = ((0, 104), (104, 96))
NC = 2
NS = 16
NW = NC * NS
BPW = B // NW
CB = 16
NCHUNK = BPW // CB
VECS = D // 16


NBUF = 8
LK = NBUF - 1


def _pool_body(x_hbm, emb_hbm, out_hbm, idxA, idxB, rows0, rows1, rows2, rows3,
               rows4, rows5, rows6, rows7, outA, outB, sem0, sem1, sem2, sem3,
               sem4, sem5, sem6, sem7, semi):
    wid = lax.axis_index("s") * NC + lax.axis_index("c")
    base = wid * BPW
    rows = (rows0, rows1, rows2, rows3, rows4, rows5, rows6, rows7)
    sems = (sem0, sem1, sem2, sem3, sem4, sem5, sem6, sem7)

    def idx_copy(c, dst):
        c = jnp.minimum(c, NCHUNK - 1)
        return pltpu.async_copy(
            x_hbm.at[pl.ds(base + c * CB, CB)], dst, semi)

    def issue(idx_ref, b, buf):
        for off, sz in GS:
            pltpu.async_copy(
                emb_hbm.at[idx_ref.at[b, pl.ds(off, sz)]],
                rows[buf].at[pl.ds(off, sz)],
                sems[buf],
            )

    def wait_row(buf):
        for off, sz in GS:
            pltpu.make_async_copy(
                emb_hbm.at[pl.ds(0, sz)],
                rows[buf].at[pl.ds(off, sz)],
                sems[buf],
            ).wait()

    def reduce_row(out_v, b, buf):
        r = rows[buf]
        un = 8
        himask = jnp.full((16,), -65536, jnp.int32)

        def rbody(i, acc):
            acc = list(acc)
            for u in range(un):
                row = i * un + u
                p = (u % 2) * 4
                for d in range(2):
                    vi = plsc.bitcast(r[row, pl.ds(d * 32, 32)], jnp.int32)
                    lo = plsc.bitcast(lax.shift_left(vi, 16), jnp.float32)
                    hi = plsc.bitcast(vi & himask, jnp.float32)
                    acc[p + d] = acc[p + d] + lo
                    acc[p + d + 2] = acc[p + d + 2] + hi
            return tuple(acc)

        zero = jnp.zeros((16,), jnp.float32)
        acc = plsc.parallel_loop(0, L // un, carry=tuple(zero for _ in range(8)))(
            lambda i, a: rbody(i, a))
        for d in range(4):
            out_v[b, pl.ds(d * 16, 16)] = (acc[d] + acc[d + 4]) * (1.0 / L)

    def phase(cur_idx, nxt_idx, c_cur, idx_handle, out_v):
        for b in range(CB):
            t = b + LK
            if t < CB:
                issue(cur_idx, t, t % NBUF)
            else:
                if t == CB:
                    idx_handle.wait()
                issue(nxt_idx, t - CB, t % NBUF)
            wait_row(b % NBUF)
            reduce_row(out_v, b, b % NBUF)
        pltpu.sync_copy(out_v, out_hbm.at[pl.ds((base + c_cur * CB), CB)])

    def pair_body(k, carry):
        hB = idx_copy(2 * k + 1, idxB)
        phase(idxA, idxB, 2 * k, hB, outA)
        hA = idx_copy(2 * k + 2, idxA)
        phase(idxB, idxA, 2 * k + 1, hA, outB)
        return carry

    pltpu.sync_copy(x_hbm.at[pl.ds(base, CB)], idxA)
    for p in range(LK):
        issue(idxA, p, p % NBUF)

    lax.fori_loop(0, NCHUNK // 2, pair_body, 0)

    for p in range(LK):
        wait_row(p % NBUF)


def _pool(x, emb):
    mesh = plsc.VectorSubcoreMesh(core_axis_name="c", subcore_axis_name="s")
    return pl.kernel(
        _pool_body,
        out_type=jax.ShapeDtypeStruct((B, D), jnp.float32),
        mesh=mesh,
        compiler_params=pltpu.CompilerParams(use_tc_tiling_on_sc=False, needs_layout_passes=False),
        scratch_types=[
            pltpu.VMEM((CB, L), jnp.int32),
            pltpu.VMEM((CB, L), jnp.int32),
            pltpu.VMEM((L, D), jnp.bfloat16),
            pltpu.VMEM((L, D), jnp.bfloat16),
            pltpu.VMEM((L, D), jnp.bfloat16),
            pltpu.VMEM((L, D), jnp.bfloat16),
            pltpu.VMEM((L, D), jnp.bfloat16),
            pltpu.VMEM((L, D), jnp.bfloat16),
            pltpu.VMEM((L, D), jnp.bfloat16),
            pltpu.VMEM((L, D), jnp.bfloat16),
            pltpu.VMEM((CB, D), jnp.float32),
            pltpu.VMEM((CB, D), jnp.float32),
            pltpu.SemaphoreType.DMA,
            pltpu.SemaphoreType.DMA,
            pltpu.SemaphoreType.DMA,
            pltpu.SemaphoreType.DMA,
            pltpu.SemaphoreType.DMA,
            pltpu.SemaphoreType.DMA,
            pltpu.SemaphoreType.DMA,
            pltpu.SemaphoreType.DMA,
            pltpu.SemaphoreType.DMA,
        ],
    )(x, emb)


def _mlp_body(h_ref, w1_ref, b1_ref, w2_ref, b2_ref, o_ref):
    h = h_ref[...]
    z = jnp.dot(h, w1_ref[...], preferred_element_type=jnp.float32) + b1_ref[...]
    z = jnp.maximum(z, 0.0)
    y = jnp.dot(z, w2_ref[...], preferred_element_type=jnp.float32) + b2_ref[...]
    m = jnp.max(y, axis=1, keepdims=True)
    lse = m + jnp.log(jnp.sum(jnp.exp(y - m), axis=1, keepdims=True))
    o_ref[...] = y - lse


def _mlp(h, W1, b1, W2, b2):
    bm = 2048
    d_h = W1.shape[1]
    d_out = W2.shape[1]
    return pl.pallas_call(
        _mlp_body,
        grid=(B // bm,),
        in_specs=[
            pl.BlockSpec((bm, D), lambda i: (i, 0)),
            pl.BlockSpec((D, d_h), lambda i: (0, 0)),
            pl.BlockSpec((1, d_h), lambda i: (0, 0)),
            pl.BlockSpec((d_h, d_out), lambda i: (0, 0)),
            pl.BlockSpec((1, d_out), lambda i: (0, 0)),
        ],
        out_specs=pl.BlockSpec((bm, d_out), lambda i: (i, 0)),
        out_shape=jax.ShapeDtypeStruct((B, d_out), jnp.float32),
    )(h, W1, b1, W2, b2)


_COL_PERM = np.concatenate([
    np.arange(0, 32, 2), np.arange(32, 64, 2),
    np.arange(1, 32, 2), np.arange(33, 64, 2),
])


def kernel(x, emb, W1, b1, W2, b2):
    pooled = _pool(x, emb.astype(jnp.bfloat16))
    return _mlp(pooled, W1[_COL_PERM], b1.reshape(1, -1), W2, b2.reshape(1, -1))
